# baseline (speedup 1.0000x reference)
import jax
import jax.numpy as jnp
from jax import lax
from jax.experimental import pallas as pl

_TOKENS_PER_TILE = 2048

_DN_RHS_T = (((1,), (1,)), ((), ()))


def _moe_lora_tile(x_ref, wr_ref, a_ref, b2_ref, out_ref):
    x = x_ref[...]
    scores = lax.dot_general(x, wr_ref[...], _DN_RHS_T,
                             preferred_element_type=jnp.float32)
    m1 = jnp.max(scores, axis=1, keepdims=True)
    is1 = scores == m1
    s2 = jnp.where(is1, -jnp.inf, scores)
    m2 = jnp.max(s2, axis=1, keepdims=True)
    g1 = 1.0 / (1.0 + jnp.exp(m2 - m1))
    g2 = 1.0 - g1
    w = jnp.where(is1, g1, 0.0) + jnp.where(s2 == m2, g2, 0.0)

    kr = a_ref.shape[0] * a_ref.shape[1]
    a2 = a_ref[...].reshape(kr, x.shape[1]).astype(jnp.bfloat16)
    h = lax.dot_general(x.astype(jnp.bfloat16), a2, _DN_RHS_T,
                        preferred_element_type=jnp.float32)
    out_ref[...] = jnp.dot((h * w).astype(jnp.bfloat16), b2_ref[...],
                           preferred_element_type=jnp.float32)


def kernel(x, A, Bmat, Wr, alpha_over_r):
    b, s, d = x.shape
    k, r, _ = A.shape
    kr = k * r
    n_tok = b * s
    tile = _TOKENS_PER_TILE

    x2 = x.reshape(n_tok, d)
    wr_rep = jnp.repeat(Wr, r, axis=0)
    b2 = (Bmat.transpose(0, 2, 1).reshape(kr, d)
          * jnp.asarray(alpha_over_r, x.dtype)).astype(jnp.bfloat16)

    out = pl.pallas_call(
        _moe_lora_tile,
        grid=(n_tok // tile,),
        in_specs=[
            pl.BlockSpec((tile, d), lambda i: (i, 0)),
            pl.BlockSpec((kr, d), lambda i: (0, 0)),
            pl.BlockSpec((k, r, d), lambda i: (0, 0, 0)),
            pl.BlockSpec((kr, d), lambda i: (0, 0)),
        ],
        out_specs=pl.BlockSpec((tile, d), lambda i: (i, 0)),
        out_shape=jax.ShapeDtypeStruct((n_tok, d), x.dtype),
    )(x2, wr_rep, A, b2)
    return out.reshape(b, s, d)
